# baseline (speedup 1.0000x reference)
import functools

import jax
import jax.numpy as jnp
from jax import lax
from jax.experimental import pallas as pl
from jax.experimental.pallas import tpu as pltpu
from jax.experimental.pallas import tpu_sc as plsc

_NGRAM_SIZES = (2, 3, 4)
_VOCAB = 500000
_DIM = 64
_PRIME = 31

_B = 8
_T = 512
_NW = 32
_CHUNK = (_B * _T) // _NW
_CHUNKS_PER_ROW = _T // _CHUNK
_PAD = 8
_L = 16


def _body(byte_hbm, emb2_hbm, emb3_hbm, emb4_hbm, out_hbm,
          bytes_v, idx2_v, idx3_v, idx4_v, e2_v, e3_v, e4_v, out_v, sem):
    nc = 2
    wid = lax.axis_index("s") * nc + lax.axis_index("c")
    b = wid // _CHUNKS_PER_ROW
    p0 = (wid % _CHUNKS_PER_ROW) * _CHUNK

    bytes_v[pl.ds(0, _L)] = jnp.zeros((_L,), jnp.int32)
    pltpu.sync_copy(byte_hbm.at[pl.ds(b * _T, _T)], bytes_v.at[pl.ds(_PAD, _T)])

    for g in range(_CHUNK // _L):
        t0 = p0 + g * _L
        v0 = bytes_v[pl.ds(_PAD + t0, _L)]
        v1 = bytes_v[pl.ds(_PAD + t0 - 1, _L)]
        v2 = bytes_v[pl.ds(_PAD + t0 - 2, _L)]
        v3 = bytes_v[pl.ds(_PAD + t0 - 3, _L)]
        h2 = v1 * _PRIME + v0
        h3 = v2 * (_PRIME * _PRIME) + h2
        h4 = lax.rem(v3 * (_PRIME * _PRIME * _PRIME) + h3, _VOCAB)
        idx2_v[pl.ds(g * _L, _L)] = h2
        idx3_v[pl.ds(g * _L, _L)] = h3
        idx4_v[pl.ds(g * _L, _L)] = h4

    c2 = pltpu.async_copy(emb2_hbm.at[idx2_v], e2_v, sem)
    c3 = pltpu.async_copy(emb3_hbm.at[idx3_v], e3_v, sem)
    c4 = pltpu.async_copy(emb4_hbm.at[idx4_v], e4_v, sem)
    c2.wait()
    c3.wait()
    c4.wait()

    def acc_body(i, _):
        for c in range(_DIM // _L):
            sl = pl.ds(c * _L, _L)
            s = e2_v[i, sl] + e3_v[i, sl] + e4_v[i, sl]
            out_v[i, sl] = s * 0.25
        return 0

    lax.fori_loop(0, _CHUNK, acc_body, 0)

    @pl.when(p0 == 0)
    def _fixup():
        for c in range(_DIM // _L):
            sl = pl.ds(c * _L, _L)
            out_v[0, sl] = jnp.zeros((_L,), jnp.float32)
            out_v[1, sl] = e2_v[1, sl] * 0.25
            out_v[2, sl] = (e2_v[2, sl] + e3_v[2, sl]) * 0.25

    pltpu.sync_copy(out_v, out_hbm.at[b, pl.ds(p0, _CHUNK)])


@jax.jit
def kernel(byte_ids, emb_2, emb_3, emb_4):
    mesh = plsc.VectorSubcoreMesh(core_axis_name="c", subcore_axis_name="s")
    f = functools.partial(
        pl.kernel,
        mesh=mesh,
        compiler_params=pltpu.CompilerParams(use_tc_tiling_on_sc=False),
        out_type=jax.ShapeDtypeStruct((_B, _T, _DIM), jnp.float32),
        scratch_types=[
            pltpu.VMEM((_PAD + _T,), jnp.int32),
            pltpu.VMEM((_CHUNK,), jnp.int32),
            pltpu.VMEM((_CHUNK,), jnp.int32),
            pltpu.VMEM((_CHUNK,), jnp.int32),
            pltpu.VMEM((_CHUNK, _DIM), jnp.float32),
            pltpu.VMEM((_CHUNK, _DIM), jnp.float32),
            pltpu.VMEM((_CHUNK, _DIM), jnp.float32),
            pltpu.VMEM((_CHUNK, _DIM), jnp.float32),
            pltpu.SemaphoreType.DMA,
        ],
    )(_body)
    return f(byte_ids.reshape(-1), emb_2, emb_3, emb_4)
